# baseline (speedup 1.0000x reference)
import functools

import jax
import jax.numpy as jnp
from jax import lax
from jax.experimental import pallas as pl
from jax.experimental.pallas import tpu as pltpu
from jax.experimental.pallas import tpu_sc as plsc

N_NODES = 10000
N_PAD = 10240
JROW = N_NODES
N_EDGES = 320000
E_PAD = 327680
NC, NS = 2, 16
CHUNK = 128
SUB = 32
ROWS_PER_TILE = N_PAD // NS

_MESH = plsc.VectorSubcoreMesh(core_axis_name="c", subcore_axis_name="s")


@functools.partial(
    pl.kernel,
    out_type=[jax.ShapeDtypeStruct((N_PAD,), jnp.float32),
              jax.ShapeDtypeStruct((N_PAD,), jnp.float32)],
    mesh=_MESH,
    scratch_types=[
        pltpu.VMEM((CHUNK,), jnp.float32),
        pltpu.VMEM((E_PAD // NS // CHUNK, CHUNK), jnp.int32),
        pltpu.VMEM_SHARED((N_PAD,), jnp.float32),
    ],
)
def _deg_kernel(src_hbm, dst_hbm, ones_hbm, zeros_hbm,
                degout_hbm, degin_hbm, ones_v, idx_v, acc):
    cid = lax.axis_index("c")
    sid = lax.axis_index("s")

    @pl.when(sid == 0)
    def _():
        pltpu.sync_copy(zeros_hbm, acc)

    pltpu.sync_copy(ones_hbm, ones_v)
    plsc.subcore_barrier()

    per_tile = E_PAD // NS
    n_chunks = per_tile // CHUNK

    def scatter_ones(idx2_hbm):
        pltpu.sync_copy(idx2_hbm.at[pl.ds(sid * n_chunks, n_chunks)], idx_v)

        def body(i, carry):
            pltpu.sync_copy(ones_v, acc.at[idx_v.at[i]], add=True)
            return carry
        lax.fori_loop(0, n_chunks, body, 0)

    @pl.when(cid == 0)
    def _():
        scatter_ones(src_hbm)

    @pl.when(cid == 1)
    def _():
        scatter_ones(dst_hbm)

    plsc.subcore_barrier()

    @pl.when(sid == 0)
    def _():
        @pl.when(cid == 0)
        def _():
            pltpu.sync_copy(acc, degout_hbm)

        @pl.when(cid == 1)
        def _():
            pltpu.sync_copy(acc, degin_hbm)


def _make_agg(wide: bool):
    per_tile = E_PAD // NS if wide else E_PAD // (NC * NS)
    n_rows = per_tile // CHUNK
    IB = 40
    NB = n_rows // IB
    NSUB = IB * (CHUNK // SUB)
    NRING = 8

    def body(tableA_hbm, tableB_hbm, src_hbm, dst_hbm, zeros_hbm,
             out0_hbm, out1_hbm,
             src_all, dst_all,
             rb0, rb1, rb2, rb3, rb4, rb5, rb6, rb7,
             acc,
             g0, g1, g2, g3, g4, g5, g6, g7,
             s0, s1, s2, s3, s4, s5, s6, s7):
        rbs = [rb0, rb1, rb2, rb3, rb4, rb5, rb6, rb7]
        gs = [g0, g1, g2, g3, g4, g5, g6, g7]
        ss = [s0, s1, s2, s3, s4, s5, s6, s7]
        cid = lax.axis_index("c")
        sid = lax.axis_index("s")
        r0 = sid * ROWS_PER_TILE
        rows_sl = pl.ds(r0, ROWS_PER_TILE)
        pltpu.sync_copy(zeros_hbm.at[rows_sl], acc.at[rows_sl])
        plsc.subcore_barrier()

        def run(table_hbm, rbase):
            def idx_sl(idx_all, row, q):
                return idx_all.at[row, pl.ds(q * SUB, SUB)]

            def gather(row, q, t):
                pltpu.async_copy(
                    table_hbm.at[idx_sl(src_all, row, q)], rbs[t], gs[t])

            def scatter(row, q, t):
                pltpu.async_copy(
                    rbs[t], acc.at[idx_sl(dst_all, row, q)], ss[t], add=True)

            def drain(sem, t):
                pltpu.make_async_copy(
                    table_hbm.at[pl.ds(0, SUB)], rbs[t], sem).wait()

            def block(b, carry):
                b0 = rbase + b * IB
                pltpu.sync_copy(src_hbm.at[pl.ds(b0, IB)], src_all)
                pltpu.sync_copy(dst_hbm.at[pl.ds(b0, IB)], dst_all)
                for t in range(NRING):
                    gather(t // 4, t % 4, t)

                def cyc(k, c2):
                    j0 = NRING * k
                    for t in range(NRING):
                        j = j0 + t
                        drain(gs[t], t)
                        scatter(2 * k + t // 4, t % 4, t)
                        u = (t + 7) % NRING
                        nxt = j + 7

                        @pl.when((nxt >= NRING) & (nxt < NSUB))
                        def _():
                            drain(ss[u], u)
                            gather(2 * k + (t + 7) // 4, (t + 7) % 4, u)
                    return c2

                lax.fori_loop(0, NSUB // NRING, cyc, 0)
                for t in range(NRING):
                    drain(ss[t], t)
                return carry

            lax.fori_loop(0, NB, block, 0)

        @pl.when(cid == 0)
        def _():
            run(tableA_hbm, sid * n_rows)

        @pl.when(cid == 1)
        def _():
            run(tableB_hbm, sid * n_rows if wide else (NS + sid) * n_rows)

        plsc.subcore_barrier()

        @pl.when(cid == 0)
        def _():
            pltpu.sync_copy(acc.at[rows_sl], out0_hbm.at[rows_sl])

        @pl.when(cid == 1)
        def _():
            pltpu.sync_copy(acc.at[rows_sl], out1_hbm.at[rows_sl])

    return pl.kernel(
        body,
        out_type=[jax.ShapeDtypeStruct((N_PAD, 128), jnp.float32),
                  jax.ShapeDtypeStruct((N_PAD, 128), jnp.float32)],
        mesh=_MESH,
        scratch_types=(
            [pltpu.VMEM((IB, CHUNK), jnp.int32)] * 2
            + [pltpu.VMEM((SUB, 128), jnp.float32)] * 8
            + [pltpu.VMEM_SHARED((N_PAD, 128), jnp.float32)]
            + [pltpu.SemaphoreType.DMA] * 16
        ),
    )


_agg_wide = _make_agg(wide=True)
_agg_split = _make_agg(wide=False)


BR = 1024
_DOT = dict(preferred_element_type=jnp.float32,
            precision=lax.Precision.HIGHEST)


def _norm(d):
    return jnp.where(d > 0, lax.rsqrt(jnp.maximum(d, 1.0)), 0.0)


def _tc_first(feat, deg_out, W1):
    def body(f_ref, d_ref, w_ref, oa_ref, ob_ref):
        ns = _norm(d_ref[...])
        h = f_ref[...] * ns
        oa_ref[...] = jnp.dot(h, w_ref[..., :128], **_DOT)
        ob_ref[...] = jnp.dot(h, w_ref[..., 128:], **_DOT)

    return pl.pallas_call(
        body,
        grid=(N_PAD // BR,),
        in_specs=[pl.BlockSpec((BR, 128), lambda i: (i, 0)),
                  pl.BlockSpec((BR, 1), lambda i: (i, 0)),
                  pl.BlockSpec((128, 256), lambda i: (0, 0))],
        out_specs=[pl.BlockSpec((BR, 128), lambda i: (i, 0)),
                   pl.BlockSpec((BR, 128), lambda i: (i, 0))],
        out_shape=[jax.ShapeDtypeStruct((N_PAD, 128), jnp.float32),
                   jax.ShapeDtypeStruct((N_PAD, 128), jnp.float32)],
    )(feat, deg_out, W1)


def _tc_mid(aggL, aggR, deg_in, deg_out, bL, bR, WT, WB, two_out: bool):
    def body(al, ar, di, do_, bl, br, wt, wb, *outs):
        nd = _norm(di[...])
        ns = _norm(do_[...])
        hl = jnp.maximum(al[...] * nd + bl[...], 0.0) * ns
        hr = jnp.maximum(ar[...] * nd + br[...], 0.0) * ns
        if two_out:
            outs[0][...] = (jnp.dot(hl, wt[..., :128], **_DOT)
                            + jnp.dot(hr, wb[..., :128], **_DOT))
            outs[1][...] = (jnp.dot(hl, wt[..., 128:], **_DOT)
                            + jnp.dot(hr, wb[..., 128:], **_DOT))
        else:
            outs[0][...] = (jnp.dot(hl, wt[...], **_DOT)
                            + jnp.dot(hr, wb[...], **_DOT))

    h_out = 256 if two_out else 128
    one = [pl.BlockSpec((BR, 128), lambda i: (i, 0)),
           jax.ShapeDtypeStruct((N_PAD, 128), jnp.float32)]
    return pl.pallas_call(
        body,
        grid=(N_PAD // BR,),
        in_specs=[pl.BlockSpec((BR, 128), lambda i: (i, 0)),
                  pl.BlockSpec((BR, 128), lambda i: (i, 0)),
                  pl.BlockSpec((BR, 1), lambda i: (i, 0)),
                  pl.BlockSpec((BR, 1), lambda i: (i, 0)),
                  pl.BlockSpec((1, 128), lambda i: (0, 0)),
                  pl.BlockSpec((1, 128), lambda i: (0, 0)),
                  pl.BlockSpec((128, h_out), lambda i: (0, 0)),
                  pl.BlockSpec((128, h_out), lambda i: (0, 0))],
        out_specs=[one[0]] * (2 if two_out else 1),
        out_shape=[one[1]] * (2 if two_out else 1),
    )(aggL, aggR, deg_in, deg_out, bL, bR, WT, WB)


def _tc_epilogue(pa, pb, deg_in, b3):
    def body(a_ref, b_ref, di, bias, o_ref):
        nd = _norm(di[...])
        o_ref[...] = (a_ref[...] + b_ref[...]) * nd + bias[...]

    return pl.pallas_call(
        body,
        grid=(N_PAD // BR,),
        in_specs=[pl.BlockSpec((BR, 128), lambda i: (i, 0)),
                  pl.BlockSpec((BR, 128), lambda i: (i, 0)),
                  pl.BlockSpec((BR, 1), lambda i: (i, 0)),
                  pl.BlockSpec((1, 128), lambda i: (0, 0))],
        out_specs=pl.BlockSpec((BR, 128), lambda i: (i, 0)),
        out_shape=jax.ShapeDtypeStruct((N_PAD, 128), jnp.float32),
    )(pa, pb, deg_in, b3)


def kernel(features, edge_index, W1, b1, W2, b2, W3, b3):
    src = edge_index[0]
    dst = edge_index[1]
    pad = JROW + jnp.arange(E_PAD - N_EDGES, dtype=jnp.int32) % (N_PAD - N_NODES)
    src_f = jnp.concatenate([src, pad])
    dst_f = jnp.concatenate([dst, pad])
    src_p = src_f.reshape(E_PAD // CHUNK, CHUNK)
    dst_p = dst_f.reshape(E_PAD // CHUNK, CHUNK)
    feat_p = jnp.zeros((N_PAD, 128), features.dtype).at[:N_NODES].set(features)
    ones_c = jnp.ones((CHUNK,), jnp.float32)
    zeros_n = jnp.zeros((N_PAD,), jnp.float32)
    zeros_nw = jnp.zeros((N_PAD, 128), jnp.float32)

    deg_out, deg_in = _deg_kernel(src_p, dst_p, ones_c, zeros_n)
    do2 = deg_out.reshape(N_PAD, 1)
    di2 = deg_in.reshape(N_PAD, 1)

    h1A, h1B = _tc_first(feat_p, do2, W1)
    aggL, aggR = _agg_wide(h1A, h1B, src_p, dst_p, zeros_nw)
    h2A, h2B = _tc_mid(aggL, aggR, di2, do2, b1[:128][None, :],
                       b1[128:][None, :], W2[:128], W2[128:], True)
    aggL2, aggR2 = _agg_wide(h2A, h2B, src_p, dst_p, zeros_nw)
    (hW3,) = _tc_mid(aggL2, aggR2, di2, do2, b2[:128][None, :],
                     b2[128:][None, :], W3[:128], W3[128:], False)
    pa, pb = _agg_split(hW3, hW3, src_p, dst_p, zeros_nw)
    out = _tc_epilogue(pa, pb, di2, b3[None, :])
    return out[:N_NODES]


# speedup vs baseline: 1.0019x; 1.0019x over previous
import functools

import jax
import jax.numpy as jnp
from jax import lax
from jax.experimental import pallas as pl
from jax.experimental.pallas import tpu as pltpu
from jax.experimental.pallas import tpu_sc as plsc

N_NODES = 10000
N_PAD = 10240
JROW = N_NODES
N_EDGES = 320000
E_PAD = 327680
NC, NS = 2, 16
CHUNK = 128
SUB = 32
ROWS_PER_TILE = N_PAD // NS

_MESH = plsc.VectorSubcoreMesh(core_axis_name="c", subcore_axis_name="s")


@functools.partial(
    pl.kernel,
    out_type=[jax.ShapeDtypeStruct((N_PAD,), jnp.float32),
              jax.ShapeDtypeStruct((N_PAD,), jnp.float32)],
    mesh=_MESH,
    scratch_types=[
        pltpu.VMEM((CHUNK,), jnp.float32),
        pltpu.VMEM((E_PAD // NS // CHUNK, CHUNK), jnp.int32),
        pltpu.VMEM_SHARED((N_PAD,), jnp.float32),
    ],
)
def _deg_kernel(src_hbm, dst_hbm, ones_hbm, zeros_hbm,
                degout_hbm, degin_hbm, ones_v, idx_v, acc):
    cid = lax.axis_index("c")
    sid = lax.axis_index("s")

    @pl.when(sid == 0)
    def _():
        pltpu.sync_copy(zeros_hbm, acc)

    pltpu.sync_copy(ones_hbm, ones_v)
    plsc.subcore_barrier()

    per_tile = E_PAD // NS
    n_chunks = per_tile // CHUNK

    def scatter_ones(idx2_hbm):
        pltpu.sync_copy(idx2_hbm.at[pl.ds(sid * n_chunks, n_chunks)], idx_v)

        def body(i, carry):
            pltpu.sync_copy(ones_v, acc.at[idx_v.at[i]], add=True)
            return carry
        lax.fori_loop(0, n_chunks, body, 0)

    @pl.when(cid == 0)
    def _():
        scatter_ones(src_hbm)

    @pl.when(cid == 1)
    def _():
        scatter_ones(dst_hbm)

    plsc.subcore_barrier()

    @pl.when(sid == 0)
    def _():
        @pl.when(cid == 0)
        def _():
            pltpu.sync_copy(acc, degout_hbm)

        @pl.when(cid == 1)
        def _():
            pltpu.sync_copy(acc, degin_hbm)


def _make_agg(wide: bool):
    per_tile = E_PAD // NS if wide else E_PAD // (NC * NS)
    n_rows = per_tile // CHUNK
    IB = 40
    NB = n_rows // IB
    NSUB = IB * (CHUNK // SUB)
    NRING = 8

    def body(tableA_hbm, tableB_hbm, src_hbm, dst_hbm, zeros_hbm,
             out0_hbm, out1_hbm,
             src_all, dst_all,
             rb0, rb1, rb2, rb3, rb4, rb5, rb6, rb7,
             acc,
             g0, g1, g2, g3, g4, g5, g6, g7,
             s0, s1, s2, s3, s4, s5, s6, s7):
        rbs = [rb0, rb1, rb2, rb3, rb4, rb5, rb6, rb7]
        gs = [g0, g1, g2, g3, g4, g5, g6, g7]
        ss = [s0, s1, s2, s3, s4, s5, s6, s7]
        cid = lax.axis_index("c")
        sid = lax.axis_index("s")
        r0 = sid * ROWS_PER_TILE
        rows_sl = pl.ds(r0, ROWS_PER_TILE)
        pltpu.sync_copy(zeros_hbm.at[rows_sl], acc.at[rows_sl])
        plsc.subcore_barrier()

        def run(table_hbm, rbase):
            def idx_sl(idx_all, row, q):
                return idx_all.at[row, pl.ds(q * SUB, SUB)]

            def gather(row, q, t):
                pltpu.async_copy(
                    table_hbm.at[idx_sl(src_all, row, q)], rbs[t], gs[t])

            def scatter(row, q, t):
                pltpu.async_copy(
                    rbs[t], acc.at[idx_sl(dst_all, row, q)], ss[t], add=True)

            def drain(sem, t):
                pltpu.make_async_copy(
                    table_hbm.at[pl.ds(0, SUB)], rbs[t], sem).wait()

            def block(b, carry):
                b0 = rbase + b * IB
                pltpu.sync_copy(src_hbm.at[pl.ds(b0, IB)], src_all)
                pltpu.sync_copy(dst_hbm.at[pl.ds(b0, IB)], dst_all)
                for t in range(NRING):
                    gather(t // 4, t % 4, t)

                def cyc(k, c2):
                    j0 = NRING * k
                    for t in range(NRING):
                        j = j0 + t
                        drain(gs[t], t)
                        scatter(2 * k + t // 4, t % 4, t)
                        u = (t + 6) % NRING
                        nxt = j + 6

                        @pl.when((nxt >= NRING) & (nxt < NSUB))
                        def _():
                            drain(ss[u], u)
                            gather(2 * k + (t + 6) // 4, (t + 6) % 4, u)
                    return c2

                lax.fori_loop(0, NSUB // NRING, cyc, 0)
                for t in range(NRING):
                    drain(ss[t], t)
                return carry

            lax.fori_loop(0, NB, block, 0)

        @pl.when(cid == 0)
        def _():
            run(tableA_hbm, sid * n_rows)

        @pl.when(cid == 1)
        def _():
            run(tableB_hbm, sid * n_rows if wide else (NS + sid) * n_rows)

        plsc.subcore_barrier()

        @pl.when(cid == 0)
        def _():
            pltpu.sync_copy(acc.at[rows_sl], out0_hbm.at[rows_sl])

        @pl.when(cid == 1)
        def _():
            pltpu.sync_copy(acc.at[rows_sl], out1_hbm.at[rows_sl])

    return pl.kernel(
        body,
        out_type=[jax.ShapeDtypeStruct((N_PAD, 128), jnp.float32),
                  jax.ShapeDtypeStruct((N_PAD, 128), jnp.float32)],
        mesh=_MESH,
        scratch_types=(
            [pltpu.VMEM((IB, CHUNK), jnp.int32)] * 2
            + [pltpu.VMEM((SUB, 128), jnp.float32)] * 8
            + [pltpu.VMEM_SHARED((N_PAD, 128), jnp.float32)]
            + [pltpu.SemaphoreType.DMA] * 16
        ),
    )


_agg_wide = _make_agg(wide=True)
_agg_split = _make_agg(wide=False)


BR = 1024
_DOT = dict(preferred_element_type=jnp.float32,
            precision=lax.Precision.HIGHEST)


def _norm(d):
    return jnp.where(d > 0, lax.rsqrt(jnp.maximum(d, 1.0)), 0.0)


def _tc_first(feat, deg_out, W1):
    def body(f_ref, d_ref, w_ref, oa_ref, ob_ref):
        ns = _norm(d_ref[...])
        h = f_ref[...] * ns
        oa_ref[...] = jnp.dot(h, w_ref[..., :128], **_DOT)
        ob_ref[...] = jnp.dot(h, w_ref[..., 128:], **_DOT)

    return pl.pallas_call(
        body,
        grid=(N_PAD // BR,),
        in_specs=[pl.BlockSpec((BR, 128), lambda i: (i, 0)),
                  pl.BlockSpec((BR, 1), lambda i: (i, 0)),
                  pl.BlockSpec((128, 256), lambda i: (0, 0))],
        out_specs=[pl.BlockSpec((BR, 128), lambda i: (i, 0)),
                   pl.BlockSpec((BR, 128), lambda i: (i, 0))],
        out_shape=[jax.ShapeDtypeStruct((N_PAD, 128), jnp.float32),
                   jax.ShapeDtypeStruct((N_PAD, 128), jnp.float32)],
    )(feat, deg_out, W1)


def _tc_mid(aggL, aggR, deg_in, deg_out, bL, bR, WT, WB, two_out: bool):
    def body(al, ar, di, do_, bl, br, wt, wb, *outs):
        nd = _norm(di[...])
        ns = _norm(do_[...])
        hl = jnp.maximum(al[...] * nd + bl[...], 0.0) * ns
        hr = jnp.maximum(ar[...] * nd + br[...], 0.0) * ns
        if two_out:
            outs[0][...] = (jnp.dot(hl, wt[..., :128], **_DOT)
                            + jnp.dot(hr, wb[..., :128], **_DOT))
            outs[1][...] = (jnp.dot(hl, wt[..., 128:], **_DOT)
                            + jnp.dot(hr, wb[..., 128:], **_DOT))
        else:
            outs[0][...] = (jnp.dot(hl, wt[...], **_DOT)
                            + jnp.dot(hr, wb[...], **_DOT))

    h_out = 256 if two_out else 128
    one = [pl.BlockSpec((BR, 128), lambda i: (i, 0)),
           jax.ShapeDtypeStruct((N_PAD, 128), jnp.float32)]
    return pl.pallas_call(
        body,
        grid=(N_PAD // BR,),
        in_specs=[pl.BlockSpec((BR, 128), lambda i: (i, 0)),
                  pl.BlockSpec((BR, 128), lambda i: (i, 0)),
                  pl.BlockSpec((BR, 1), lambda i: (i, 0)),
                  pl.BlockSpec((BR, 1), lambda i: (i, 0)),
                  pl.BlockSpec((1, 128), lambda i: (0, 0)),
                  pl.BlockSpec((1, 128), lambda i: (0, 0)),
                  pl.BlockSpec((128, h_out), lambda i: (0, 0)),
                  pl.BlockSpec((128, h_out), lambda i: (0, 0))],
        out_specs=[one[0]] * (2 if two_out else 1),
        out_shape=[one[1]] * (2 if two_out else 1),
    )(aggL, aggR, deg_in, deg_out, bL, bR, WT, WB)


def _tc_epilogue(pa, pb, deg_in, b3):
    def body(a_ref, b_ref, di, bias, o_ref):
        nd = _norm(di[...])
        o_ref[...] = (a_ref[...] + b_ref[...]) * nd + bias[...]

    return pl.pallas_call(
        body,
        grid=(N_PAD // BR,),
        in_specs=[pl.BlockSpec((BR, 128), lambda i: (i, 0)),
                  pl.BlockSpec((BR, 128), lambda i: (i, 0)),
                  pl.BlockSpec((BR, 1), lambda i: (i, 0)),
                  pl.BlockSpec((1, 128), lambda i: (0, 0))],
        out_specs=pl.BlockSpec((BR, 128), lambda i: (i, 0)),
        out_shape=jax.ShapeDtypeStruct((N_PAD, 128), jnp.float32),
    )(pa, pb, deg_in, b3)


def kernel(features, edge_index, W1, b1, W2, b2, W3, b3):
    src = edge_index[0]
    dst = edge_index[1]
    pad = JROW + jnp.arange(E_PAD - N_EDGES, dtype=jnp.int32) % (N_PAD - N_NODES)
    src_f = jnp.concatenate([src, pad])
    dst_f = jnp.concatenate([dst, pad])
    src_p = src_f.reshape(E_PAD // CHUNK, CHUNK)
    dst_p = dst_f.reshape(E_PAD // CHUNK, CHUNK)
    feat_p = jnp.zeros((N_PAD, 128), features.dtype).at[:N_NODES].set(features)
    ones_c = jnp.ones((CHUNK,), jnp.float32)
    zeros_n = jnp.zeros((N_PAD,), jnp.float32)
    zeros_nw = jnp.zeros((N_PAD, 128), jnp.float32)

    deg_out, deg_in = _deg_kernel(src_p, dst_p, ones_c, zeros_n)
    do2 = deg_out.reshape(N_PAD, 1)
    di2 = deg_in.reshape(N_PAD, 1)

    h1A, h1B = _tc_first(feat_p, do2, W1)
    aggL, aggR = _agg_wide(h1A, h1B, src_p, dst_p, zeros_nw)
    h2A, h2B = _tc_mid(aggL, aggR, di2, do2, b1[:128][None, :],
                       b1[128:][None, :], W2[:128], W2[128:], True)
    aggL2, aggR2 = _agg_wide(h2A, h2B, src_p, dst_p, zeros_nw)
    (hW3,) = _tc_mid(aggL2, aggR2, di2, do2, b2[:128][None, :],
                     b2[128:][None, :], W3[:128], W3[128:], False)
    pa, pb = _agg_split(hW3, hW3, src_p, dst_p, zeros_nw)
    out = _tc_epilogue(pa, pb, di2, b3[None, :])
    return out[:N_NODES]


# speedup vs baseline: 1.0524x; 1.0504x over previous
import functools

import jax
import jax.numpy as jnp
from jax import lax
from jax.experimental import pallas as pl
from jax.experimental.pallas import tpu as pltpu
from jax.experimental.pallas import tpu_sc as plsc

N_NODES = 10000
N_PAD = 10240
JROW = N_NODES
N_EDGES = 320000
E_PAD = 327680
NC, NS = 2, 16
CHUNK = 128
SUB = 32
ROWS_PER_TILE = N_PAD // NS

_MESH = plsc.VectorSubcoreMesh(core_axis_name="c", subcore_axis_name="s")


@functools.partial(
    pl.kernel,
    out_type=[jax.ShapeDtypeStruct((N_PAD,), jnp.float32),
              jax.ShapeDtypeStruct((N_PAD,), jnp.float32)],
    mesh=_MESH,
    scratch_types=[
        pltpu.VMEM((CHUNK,), jnp.float32),
        pltpu.VMEM((E_PAD // NS // CHUNK, CHUNK), jnp.int32),
        pltpu.VMEM_SHARED((N_PAD,), jnp.float32),
    ],
)
def _deg_kernel(src_hbm, dst_hbm, ones_hbm, zeros_hbm,
                degout_hbm, degin_hbm, ones_v, idx_v, acc):
    cid = lax.axis_index("c")
    sid = lax.axis_index("s")

    @pl.when(sid == 0)
    def _():
        pltpu.sync_copy(zeros_hbm, acc)

    pltpu.sync_copy(ones_hbm, ones_v)
    plsc.subcore_barrier()

    per_tile = E_PAD // NS
    n_chunks = per_tile // CHUNK

    def scatter_ones(idx2_hbm):
        pltpu.sync_copy(idx2_hbm.at[pl.ds(sid * n_chunks, n_chunks)], idx_v)

        def body(i, carry):
            pltpu.sync_copy(ones_v, acc.at[idx_v.at[i]], add=True)
            return carry
        lax.fori_loop(0, n_chunks, body, 0)

    @pl.when(cid == 0)
    def _():
        scatter_ones(src_hbm)

    @pl.when(cid == 1)
    def _():
        scatter_ones(dst_hbm)

    plsc.subcore_barrier()

    @pl.when(sid == 0)
    def _():
        @pl.when(cid == 0)
        def _():
            pltpu.sync_copy(acc, degout_hbm)

        @pl.when(cid == 1)
        def _():
            pltpu.sync_copy(acc, degin_hbm)


def _make_agg(wide: bool):
    per_tile = E_PAD // NS if wide else E_PAD // (NC * NS)
    n_rows = per_tile // CHUNK
    IB = 40
    NB = n_rows // IB
    NSUB = IB * (CHUNK // SUB)
    NRING = 8

    def body(tableA_hbm, tableB_hbm, src_hbm, dst_hbm, zeros_hbm,
             out0_hbm, out1_hbm,
             src_all, dst_all,
             rb0, rb1, rb2, rb3, rb4, rb5, rb6, rb7,
             acc,
             g0, g1, g2, g3, g4, g5, g6, g7,
             s0, s1, s2, s3, s4, s5, s6, s7):
        rbs = [rb0, rb1, rb2, rb3, rb4, rb5, rb6, rb7]
        gs = [g0, g1, g2, g3, g4, g5, g6, g7]
        ss = [s0, s1, s2, s3, s4, s5, s6, s7]
        cid = lax.axis_index("c")
        sid = lax.axis_index("s")
        r0 = sid * ROWS_PER_TILE
        rows_sl = pl.ds(r0, ROWS_PER_TILE)
        pltpu.sync_copy(zeros_hbm.at[rows_sl], acc.at[rows_sl])
        plsc.subcore_barrier()

        def run(table_hbm, rbase):
            def idx_sl(idx_all, row, q):
                return idx_all.at[row, pl.ds(q * SUB, SUB)]

            def gather(row, q, t):
                pltpu.async_copy(
                    table_hbm.at[idx_sl(src_all, row, q)], rbs[t], gs[t])

            def scatter(row, q, t):
                pltpu.async_copy(
                    rbs[t], acc.at[idx_sl(dst_all, row, q)], ss[t], add=True)

            def drain(sem, t):
                pltpu.make_async_copy(
                    table_hbm.at[pl.ds(0, SUB)], rbs[t], sem).wait()

            def block(b, carry):
                b0 = rbase + b * IB
                pltpu.sync_copy(src_hbm.at[pl.ds(b0, IB)], src_all)
                pltpu.sync_copy(dst_hbm.at[pl.ds(b0, IB)], dst_all)
                for t in range(NRING):
                    gather(t // 4, t % 4, t)

                def cyc(k, c2):
                    j0 = NRING * k
                    for t in range(NRING):
                        j = j0 + t
                        drain(gs[t], t)
                        scatter(2 * k + t // 4, t % 4, t)
                        u = (t + 6) % NRING
                        nxt = j + 6

                        @pl.when((nxt >= NRING) & (nxt < NSUB))
                        def _():
                            drain(ss[u], u)
                            gather(2 * k + (t + 6) // 4, (t + 6) % 4, u)
                    return c2

                lax.fori_loop(0, NSUB // NRING, cyc, 0)
                for t in range(NRING):
                    drain(ss[t], t)
                return carry

            lax.fori_loop(0, NB, block, 0)

        @pl.when(cid == 0)
        def _():
            run(tableA_hbm, sid * n_rows)

        @pl.when(cid == 1)
        def _():
            run(tableB_hbm, sid * n_rows if wide else (NS + sid) * n_rows)

        plsc.subcore_barrier()

        @pl.when(cid == 0)
        def _():
            pltpu.sync_copy(acc.at[rows_sl], out0_hbm.at[rows_sl])

        @pl.when(cid == 1)
        def _():
            pltpu.sync_copy(acc.at[rows_sl], out1_hbm.at[rows_sl])

    return pl.kernel(
        body,
        out_type=[jax.ShapeDtypeStruct((N_PAD, 128), jnp.float32),
                  jax.ShapeDtypeStruct((N_PAD, 128), jnp.float32)],
        mesh=_MESH,
        scratch_types=(
            [pltpu.VMEM((IB, CHUNK), jnp.int32)] * 2
            + [pltpu.VMEM((SUB, 128), jnp.float32)] * 8
            + [pltpu.VMEM_SHARED((N_PAD, 128), jnp.float32)]
            + [pltpu.SemaphoreType.DMA] * 16
        ),
    )


_agg_wide = _make_agg(wide=True)
_agg_split = _make_agg(wide=False)


BR = 2560
_DOT = dict(preferred_element_type=jnp.float32)


def _norm(d):
    return jnp.where(d > 0, lax.rsqrt(jnp.maximum(d, 1.0)), 0.0)


def _tc_first(feat, deg_out, W1):
    def body(f_ref, d_ref, w_ref, oa_ref, ob_ref):
        ns = _norm(d_ref[...])
        h = f_ref[...] * ns
        oa_ref[...] = jnp.dot(h, w_ref[..., :128], **_DOT)
        ob_ref[...] = jnp.dot(h, w_ref[..., 128:], **_DOT)

    return pl.pallas_call(
        body,
        grid=(N_PAD // BR,),
        in_specs=[pl.BlockSpec((BR, 128), lambda i: (i, 0)),
                  pl.BlockSpec((BR, 1), lambda i: (i, 0)),
                  pl.BlockSpec((128, 256), lambda i: (0, 0))],
        out_specs=[pl.BlockSpec((BR, 128), lambda i: (i, 0)),
                   pl.BlockSpec((BR, 128), lambda i: (i, 0))],
        out_shape=[jax.ShapeDtypeStruct((N_PAD, 128), jnp.float32),
                   jax.ShapeDtypeStruct((N_PAD, 128), jnp.float32)],
    )(feat, deg_out, W1)


def _tc_mid(aggL, aggR, deg_in, deg_out, bL, bR, WT, WB, two_out: bool):
    def body(al, ar, di, do_, bl, br, wt, wb, *outs):
        nd = _norm(di[...])
        ns = _norm(do_[...])
        hl = jnp.maximum(al[...] * nd + bl[...], 0.0) * ns
        hr = jnp.maximum(ar[...] * nd + br[...], 0.0) * ns
        if two_out:
            outs[0][...] = (jnp.dot(hl, wt[..., :128], **_DOT)
                            + jnp.dot(hr, wb[..., :128], **_DOT))
            outs[1][...] = (jnp.dot(hl, wt[..., 128:], **_DOT)
                            + jnp.dot(hr, wb[..., 128:], **_DOT))
        else:
            outs[0][...] = (jnp.dot(hl, wt[...], **_DOT)
                            + jnp.dot(hr, wb[...], **_DOT))

    h_out = 256 if two_out else 128
    one = [pl.BlockSpec((BR, 128), lambda i: (i, 0)),
           jax.ShapeDtypeStruct((N_PAD, 128), jnp.float32)]
    return pl.pallas_call(
        body,
        grid=(N_PAD // BR,),
        in_specs=[pl.BlockSpec((BR, 128), lambda i: (i, 0)),
                  pl.BlockSpec((BR, 128), lambda i: (i, 0)),
                  pl.BlockSpec((BR, 1), lambda i: (i, 0)),
                  pl.BlockSpec((BR, 1), lambda i: (i, 0)),
                  pl.BlockSpec((1, 128), lambda i: (0, 0)),
                  pl.BlockSpec((1, 128), lambda i: (0, 0)),
                  pl.BlockSpec((128, h_out), lambda i: (0, 0)),
                  pl.BlockSpec((128, h_out), lambda i: (0, 0))],
        out_specs=[one[0]] * (2 if two_out else 1),
        out_shape=[one[1]] * (2 if two_out else 1),
    )(aggL, aggR, deg_in, deg_out, bL, bR, WT, WB)


def _tc_epilogue(pa, pb, deg_in, b3):
    eb = 1000

    def body(a_ref, b_ref, di, bias, o_ref):
        nd = _norm(di[...])
        o_ref[...] = (a_ref[...] + b_ref[...]) * nd + bias[...]

    return pl.pallas_call(
        body,
        grid=(N_NODES // eb,),
        in_specs=[pl.BlockSpec((eb, 128), lambda i: (i, 0)),
                  pl.BlockSpec((eb, 128), lambda i: (i, 0)),
                  pl.BlockSpec((eb, 1), lambda i: (i, 0)),
                  pl.BlockSpec((1, 128), lambda i: (0, 0))],
        out_specs=pl.BlockSpec((eb, 128), lambda i: (i, 0)),
        out_shape=jax.ShapeDtypeStruct((N_NODES, 128), jnp.float32),
    )(pa, pb, deg_in, b3)


def kernel(features, edge_index, W1, b1, W2, b2, W3, b3):
    src = edge_index[0]
    dst = edge_index[1]
    pad = JROW + jnp.arange(E_PAD - N_EDGES, dtype=jnp.int32) % (N_PAD - N_NODES)
    src_f = jnp.concatenate([src, pad])
    dst_f = jnp.concatenate([dst, pad])
    src_p = src_f.reshape(E_PAD // CHUNK, CHUNK)
    dst_p = dst_f.reshape(E_PAD // CHUNK, CHUNK)
    feat_p = jnp.zeros((N_PAD, 128), features.dtype).at[:N_NODES].set(features)
    ones_c = jnp.ones((CHUNK,), jnp.float32)
    zeros_n = jnp.zeros((N_PAD,), jnp.float32)
    zeros_nw = jnp.zeros((N_PAD, 128), jnp.float32)

    deg_out, deg_in = _deg_kernel(src_p, dst_p, ones_c, zeros_n)
    do2 = deg_out.reshape(N_PAD, 1)
    di2 = deg_in.reshape(N_PAD, 1)

    h1A, h1B = _tc_first(feat_p, do2, W1)
    aggL, aggR = _agg_wide(h1A, h1B, src_p, dst_p, zeros_nw)
    h2A, h2B = _tc_mid(aggL, aggR, di2, do2, b1[:128][None, :],
                       b1[128:][None, :], W2[:128], W2[128:], True)
    aggL2, aggR2 = _agg_wide(h2A, h2B, src_p, dst_p, zeros_nw)
    (hW3,) = _tc_mid(aggL2, aggR2, di2, do2, b2[:128][None, :],
                     b2[128:][None, :], W3[:128], W3[128:], False)
    pa, pb = _agg_split(hW3, hW3, src_p, dst_p, zeros_nw)
    return _tc_epilogue(pa, pb, di2, b3[None, :])
